# baseline (speedup 1.0000x reference)
import jax
import jax.numpy as jnp
from jax.experimental import pallas as pl
from jax.experimental.pallas import tpu as pltpu
from jax.experimental.pallas import tpu_sc as plsc

HIDDEN = 2048
FC = 512
CODE = 64
CB = 8192
BATCH = 8192

_INV_B = 1.0 / BATCH

TM1 = 1024
TMV = 2048
TM2 = 512
NC = 2048
TM4 = 2048
TM5 = 1024
GW = 256


def _mm(a, b):
    return jax.lax.dot_general(a.astype(jnp.bfloat16), b.astype(jnp.bfloat16),
                               (((1,), (1,)), ((), ())),
                               preferred_element_type=jnp.float32)


def _enc1_body(h_ref, w_ref, cb_ref, x_ref, s_ref, cn_ref, cbp_ref):
    i = pl.program_id(0)
    x = _mm(h_ref[...], w_ref[...])
    x_ref[...] = x

    @pl.when(i == 0)
    def _():
        s_ref[...] = jnp.zeros_like(s_ref)
        cb = cb_ref[...]
        cn_ref[...] = jnp.sum(cb * cb, axis=1, keepdims=True)
        cbp_ref[...] = jnp.concatenate([cb, jnp.zeros_like(cb)], axis=1)

    s_ref[...] += jnp.sum(x, axis=0, keepdims=True)


def _encode1(h, w, cb):
    return pl.pallas_call(
        _enc1_body,
        grid=(BATCH // TM1,),
        in_specs=[
            pl.BlockSpec((TM1, HIDDEN), lambda i: (i, 0)),
            pl.BlockSpec((FC, HIDDEN), lambda i: (0, 0)),
            pl.BlockSpec((CB, CODE), lambda i: (0, 0)),
        ],
        out_specs=[
            pl.BlockSpec((TM1, FC), lambda i: (i, 0)),
            pl.BlockSpec((1, FC), lambda i: (0, 0)),
            pl.BlockSpec((CB, 1), lambda i: (0, 0)),
            pl.BlockSpec((CB, 2 * CODE), lambda i: (0, 0)),
        ],
        out_shape=[
            jax.ShapeDtypeStruct((BATCH, FC), jnp.float32),
            jax.ShapeDtypeStruct((1, FC), jnp.float32),
            jax.ShapeDtypeStruct((CB, 1), jnp.float32),
            jax.ShapeDtypeStruct((CB, 2 * CODE), jnp.float32),
        ],
    )(h, w, cb)


def _var_body(x_ref, s_ref, v_ref):
    i = pl.program_id(0)

    @pl.when(i == 0)
    def _():
        v_ref[...] = jnp.zeros_like(v_ref)

    mean = s_ref[...] * _INV_B
    c = x_ref[...] - mean
    v_ref[...] += jnp.sum(c * c, axis=0, keepdims=True)


def _var_pass(x, s):
    return pl.pallas_call(
        _var_body,
        grid=(BATCH // TMV,),
        in_specs=[
            pl.BlockSpec((TMV, FC), lambda i: (i, 0)),
            pl.BlockSpec((1, FC), lambda i: (0, 0)),
        ],
        out_specs=pl.BlockSpec((1, FC), lambda i: (0, 0)),
        out_shape=jax.ShapeDtypeStruct((1, FC), jnp.float32),
    )(x, s)


def _enc2_body(x_ref, s_ref, v_ref, g_ref, b_ref, w2_ref, b2_ref, cb_ref,
               cn_ref, ze_ref, idx_ref, idxrow_ref):
    mean = s_ref[...] * _INV_B
    var = v_ref[...] * _INV_B
    xn = g_ref[...] * (x_ref[...] - mean) * jax.lax.rsqrt(var + 1e-5) \
        + b_ref[...]
    xn = jnp.where(xn >= 0, xn, 0.01 * xn)
    ze = _mm(xn, w2_ref[...]) + b2_ref[...]
    ze_ref[...] = ze
    zn = jnp.sum(ze * ze, axis=1, keepdims=True)

    best_m = jnp.full((TM2, 1), jnp.inf, jnp.float32)
    best_a = jnp.zeros((TM2, 1), jnp.int32)
    for c in range(CB // NC):
        sc = _mm(ze, cb_ref[c * NC:(c + 1) * NC, :])
        s = (zn - 2.0 * sc) + cn_ref[:, c * NC:(c + 1) * NC]
        mc = jnp.min(s, axis=1, keepdims=True)
        iota = jax.lax.broadcasted_iota(jnp.int32, s.shape, 1) + c * NC
        ac = jnp.min(jnp.where(s == mc, iota, CB), axis=1, keepdims=True)
        take = mc < best_m
        best_m = jnp.where(take, mc.astype(jnp.bfloat16).astype(jnp.float32),
                           best_m)
        best_a = jnp.where(take, ac, best_a)
    idx_ref[...] = best_a
    idxrow_ref[...] = best_a.reshape(1, TM2)


def _encode2(x, xs, xv, g, b, w2, b2, cb, cn_row):
    return pl.pallas_call(
        _enc2_body,
        grid=(BATCH // TM2,),
        in_specs=[
            pl.BlockSpec((TM2, FC), lambda i: (i, 0)),
            pl.BlockSpec((1, FC), lambda i: (0, 0)),
            pl.BlockSpec((1, FC), lambda i: (0, 0)),
            pl.BlockSpec((1, FC), lambda i: (0, 0)),
            pl.BlockSpec((1, FC), lambda i: (0, 0)),
            pl.BlockSpec((CODE, FC), lambda i: (0, 0)),
            pl.BlockSpec((1, CODE), lambda i: (0, 0)),
            pl.BlockSpec((CB, CODE), lambda i: (0, 0)),
            pl.BlockSpec((1, CB), lambda i: (0, 0)),
        ],
        out_specs=[
            pl.BlockSpec((TM2, CODE), lambda i: (i, 0)),
            pl.BlockSpec((TM2, 1), lambda i: (i, 0)),
            pl.BlockSpec((1, TM2), lambda i: (0, i)),
        ],
        out_shape=[
            jax.ShapeDtypeStruct((BATCH, CODE), jnp.float32),
            jax.ShapeDtypeStruct((BATCH, 1), jnp.int32),
            jax.ShapeDtypeStruct((1, BATCH), jnp.int32),
        ],
    )(x, xs, xv, g, b, w2, b2, cb, cn_row)


def _gather_zq(cb_padded, idx_row):
    @pl.kernel(out_type=jax.ShapeDtypeStruct((BATCH, 2 * CODE), jnp.float32),
               mesh=plsc.VectorSubcoreMesh(core_axis_name="c",
                                           subcore_axis_name="s"))
    def kern(cb_hbm, i_hbm, o_hbm):
        def body(i_vmem, o_vmem):
            pltpu.sync_copy(cb_hbm.at[i_vmem.at[0]], o_vmem)

        pltpu.emit_pipeline(
            body,
            grid=(BATCH // GW,),
            in_specs=[pl.BlockSpec((1, GW), index_map=lambda i: (0, i))],
            out_specs=[pl.BlockSpec((GW, 2 * CODE),
                                    index_map=lambda i: (i, 0))],
            core_axis_name=("c", "s"),
            dimension_semantics=(pltpu.PARALLEL,),
        )(i_hbm, o_hbm)

    return kern(cb_padded, idx_row)


def _dec1_body(zqp_ref, w_ref, y_ref, s_ref, q_ref, zq_ref):
    i = pl.program_id(0)
    zq = zqp_ref[:, :CODE]
    zq_ref[...] = zq
    y = _mm(zq, w_ref[...])
    y_ref[...] = y

    @pl.when(i == 0)
    def _():
        s_ref[...] = jnp.zeros_like(s_ref)
        q_ref[...] = jnp.zeros_like(q_ref)

    s_ref[...] += jnp.sum(y, axis=0, keepdims=True)
    q_ref[...] += jnp.sum(y * y, axis=0, keepdims=True)


def _decode1(zqp, w):
    return pl.pallas_call(
        _dec1_body,
        grid=(BATCH // TM4,),
        in_specs=[
            pl.BlockSpec((TM4, 2 * CODE), lambda i: (i, 0)),
            pl.BlockSpec((FC, CODE), lambda i: (0, 0)),
        ],
        out_specs=[
            pl.BlockSpec((TM4, FC), lambda i: (i, 0)),
            pl.BlockSpec((1, FC), lambda i: (0, 0)),
            pl.BlockSpec((1, FC), lambda i: (0, 0)),
            pl.BlockSpec((TM4, CODE), lambda i: (i, 0)),
        ],
        out_shape=[
            jax.ShapeDtypeStruct((BATCH, FC), jnp.float32),
            jax.ShapeDtypeStruct((1, FC), jnp.float32),
            jax.ShapeDtypeStruct((1, FC), jnp.float32),
            jax.ShapeDtypeStruct((BATCH, CODE), jnp.float32),
        ],
    )(zqp, w)


def _dec2_body(y_ref, s_ref, q_ref, g_ref, b_ref, w_ref, b2_ref, o_ref):
    mean = s_ref[...] * _INV_B
    var = q_ref[...] * _INV_B - mean * mean
    yn = g_ref[...] * (y_ref[...] - mean) * jax.lax.rsqrt(var + 1e-5) \
        + b_ref[...]
    yn = jnp.where(yn >= 0, yn, 0.01 * yn)
    o_ref[...] = _mm(yn, w_ref[...]) + b2_ref[...]


def _decode2(y, ys, yq, g, b, w, b2):
    return pl.pallas_call(
        _dec2_body,
        grid=(BATCH // TM5,),
        in_specs=[
            pl.BlockSpec((TM5, FC), lambda i: (i, 0)),
            pl.BlockSpec((1, FC), lambda i: (0, 0)),
            pl.BlockSpec((1, FC), lambda i: (0, 0)),
            pl.BlockSpec((1, FC), lambda i: (0, 0)),
            pl.BlockSpec((1, FC), lambda i: (0, 0)),
            pl.BlockSpec((HIDDEN, FC), lambda i: (0, 0)),
            pl.BlockSpec((1, HIDDEN), lambda i: (0, 0)),
        ],
        out_specs=pl.BlockSpec((TM5, HIDDEN), lambda i: (i, 0)),
        out_shape=jax.ShapeDtypeStruct((BATCH, HIDDEN), jnp.float32),
    )(y, ys, yq, g, b, w, b2)


def kernel(h, W_e1, g_e1, be_e1, W_e2, b_e2, W_d1, g_d1, be_d1, W_d2, b_d2,
           codebook):
    x, xs, cn_col, cbp = _encode1(h, W_e1, codebook)
    xv = _var_pass(x, xs)
    ze, idx2, idxrow = _encode2(x, xs, xv, g_e1.reshape(1, -1),
                                be_e1.reshape(1, -1), W_e2,
                                b_e2.reshape(1, -1), codebook,
                                cn_col.reshape(1, -1))
    zqp = _gather_zq(cbp, idxrow)
    y, ys, yq, zq = _decode1(zqp, W_d1)
    h_recon = _decode2(y, ys, yq, g_d1.reshape(1, -1), be_d1.reshape(1, -1),
                       W_d2, b_d2.reshape(1, -1))
    return ze, zq, h_recon, idx2.reshape(-1)


# speedup vs baseline: 1.0525x; 1.0525x over previous
import jax
import jax.numpy as jnp
from jax.experimental import pallas as pl
from jax.experimental.pallas import tpu as pltpu
from jax.experimental.pallas import tpu_sc as plsc

HIDDEN = 2048
FC = 512
CODE = 64
CB = 8192
BATCH = 8192

_INV_B = 1.0 / BATCH

TM1 = 1024
TMV = 2048
TM2 = 512
NC = 2048
TM4 = 2048
TM5 = 1024
GW = 256


def _mm(a, b):
    return jax.lax.dot_general(a.astype(jnp.bfloat16), b.astype(jnp.bfloat16),
                               (((1,), (1,)), ((), ())),
                               preferred_element_type=jnp.float32)


def _enc1_body(h_ref, w_ref, cb_ref, x_ref, s_ref, cn_ref, cbp_ref):
    i = pl.program_id(0)
    x = _mm(h_ref[...], w_ref[...])
    x_ref[...] = x

    @pl.when(i == 0)
    def _():
        s_ref[...] = jnp.zeros_like(s_ref)
        cb = cb_ref[...]
        cn_ref[...] = jnp.sum(cb * cb, axis=1, keepdims=True)
        cbp_ref[...] = jnp.concatenate([cb, jnp.zeros_like(cb)], axis=1)

    s_ref[...] += jnp.sum(x, axis=0, keepdims=True)


def _encode1(h, w, cb):
    return pl.pallas_call(
        _enc1_body,
        grid=(BATCH // TM1,),
        in_specs=[
            pl.BlockSpec((TM1, HIDDEN), lambda i: (i, 0)),
            pl.BlockSpec((FC, HIDDEN), lambda i: (0, 0)),
            pl.BlockSpec((CB, CODE), lambda i: (0, 0)),
        ],
        out_specs=[
            pl.BlockSpec((TM1, FC), lambda i: (i, 0)),
            pl.BlockSpec((1, FC), lambda i: (0, 0)),
            pl.BlockSpec((CB, 1), lambda i: (0, 0)),
            pl.BlockSpec((CB, 2 * CODE), lambda i: (0, 0)),
        ],
        out_shape=[
            jax.ShapeDtypeStruct((BATCH, FC), jnp.float32),
            jax.ShapeDtypeStruct((1, FC), jnp.float32),
            jax.ShapeDtypeStruct((CB, 1), jnp.float32),
            jax.ShapeDtypeStruct((CB, 2 * CODE), jnp.float32),
        ],
    )(h, w, cb)


def _var_body(x_ref, s_ref, v_ref):
    i = pl.program_id(0)

    @pl.when(i == 0)
    def _():
        v_ref[...] = jnp.zeros_like(v_ref)

    mean = s_ref[...] * _INV_B
    c = x_ref[...] - mean
    v_ref[...] += jnp.sum(c * c, axis=0, keepdims=True)


def _var_pass(x, s):
    return pl.pallas_call(
        _var_body,
        grid=(BATCH // TMV,),
        in_specs=[
            pl.BlockSpec((TMV, FC), lambda i: (i, 0)),
            pl.BlockSpec((1, FC), lambda i: (0, 0)),
        ],
        out_specs=pl.BlockSpec((1, FC), lambda i: (0, 0)),
        out_shape=jax.ShapeDtypeStruct((1, FC), jnp.float32),
    )(x, s)


def _enc2_body(x_ref, s_ref, v_ref, g_ref, b_ref, w2_ref, b2_ref, cb_ref,
               cn_ref, ze_ref, idx_ref, idxrow_ref):
    mean = s_ref[...] * _INV_B
    var = v_ref[...] * _INV_B
    xn = g_ref[...] * (x_ref[...] - mean) * jax.lax.rsqrt(var + 1e-5) \
        + b_ref[...]
    xn = jnp.where(xn >= 0, xn, 0.01 * xn)
    ze = _mm(xn, w2_ref[...]) + b2_ref[...]
    ze_ref[...] = ze
    zn = jnp.sum(ze * ze, axis=1, keepdims=True)

    best_m = jnp.full((TM2, 1), jnp.inf, jnp.float32)
    best_a = jnp.zeros((TM2, 1), jnp.int32)
    for c in range(CB // NC):
        sc = _mm(ze, cb_ref[c * NC:(c + 1) * NC, :])
        s = (zn - 2.0 * sc) + cn_ref[:, c * NC:(c + 1) * NC]
        mc = jnp.min(s, axis=1, keepdims=True)
        iota = jax.lax.broadcasted_iota(jnp.int32, s.shape, 1) + c * NC
        ac = jnp.min(jnp.where(s == mc, iota, CB), axis=1, keepdims=True)
        take = mc < best_m
        best_m = jnp.where(take, mc.astype(jnp.bfloat16).astype(jnp.float32),
                           best_m)
        best_a = jnp.where(take, ac, best_a)
    idx_ref[...] = best_a
    idxrow_ref[...] = best_a.reshape(1, TM2)


HALF = BATCH // 2


def _encode2(x, xs, xv, g, b, w2, b2, cb, cn_row, half):
    off = half * (HALF // TM2)
    return pl.pallas_call(
        _enc2_body,
        grid=(HALF // TM2,),
        in_specs=[
            pl.BlockSpec((TM2, FC), lambda i: (i + off, 0)),
            pl.BlockSpec((1, FC), lambda i: (0, 0)),
            pl.BlockSpec((1, FC), lambda i: (0, 0)),
            pl.BlockSpec((1, FC), lambda i: (0, 0)),
            pl.BlockSpec((1, FC), lambda i: (0, 0)),
            pl.BlockSpec((CODE, FC), lambda i: (0, 0)),
            pl.BlockSpec((1, CODE), lambda i: (0, 0)),
            pl.BlockSpec((CB, CODE), lambda i: (0, 0)),
            pl.BlockSpec((1, CB), lambda i: (0, 0)),
        ],
        out_specs=[
            pl.BlockSpec((TM2, CODE), lambda i: (i, 0)),
            pl.BlockSpec((TM2, 1), lambda i: (i, 0)),
            pl.BlockSpec((1, TM2), lambda i: (0, i)),
        ],
        out_shape=[
            jax.ShapeDtypeStruct((HALF, CODE), jnp.float32),
            jax.ShapeDtypeStruct((HALF, 1), jnp.int32),
            jax.ShapeDtypeStruct((1, HALF), jnp.int32),
        ],
    )(x, xs, xv, g, b, w2, b2, cb, cn_row)


def _gather_zq(cb_padded, idx_row):
    n = idx_row.shape[1]

    @pl.kernel(out_type=jax.ShapeDtypeStruct((n, 2 * CODE), jnp.float32),
               mesh=plsc.VectorSubcoreMesh(core_axis_name="c",
                                           subcore_axis_name="s"))
    def kern(cb_hbm, i_hbm, o_hbm):
        def body(i_vmem, o_vmem):
            pltpu.sync_copy(cb_hbm.at[i_vmem.at[0]], o_vmem)

        pltpu.emit_pipeline(
            body,
            grid=(n // GW,),
            in_specs=[pl.BlockSpec((1, GW), index_map=lambda i: (0, i))],
            out_specs=[pl.BlockSpec((GW, 2 * CODE),
                                    index_map=lambda i: (i, 0))],
            core_axis_name=("c", "s"),
            dimension_semantics=(pltpu.PARALLEL,),
        )(i_hbm, o_hbm)

    return kern(cb_padded, idx_row)


def _dec1_body(zqp_ref, w_ref, y_ref, s_ref, q_ref, zq_ref):
    i = pl.program_id(0)
    zq = zqp_ref[:, :CODE]
    zq_ref[...] = zq
    y = _mm(zq, w_ref[...])
    y_ref[...] = y

    @pl.when(i == 0)
    def _():
        s_ref[...] = jnp.zeros_like(s_ref)
        q_ref[...] = jnp.zeros_like(q_ref)

    s_ref[...] += jnp.sum(y, axis=0, keepdims=True)
    q_ref[...] += jnp.sum(y * y, axis=0, keepdims=True)


def _decode1(zqp, w):
    n = zqp.shape[0]
    return pl.pallas_call(
        _dec1_body,
        grid=(n // TM4,),
        in_specs=[
            pl.BlockSpec((TM4, 2 * CODE), lambda i: (i, 0)),
            pl.BlockSpec((FC, CODE), lambda i: (0, 0)),
        ],
        out_specs=[
            pl.BlockSpec((TM4, FC), lambda i: (i, 0)),
            pl.BlockSpec((1, FC), lambda i: (0, 0)),
            pl.BlockSpec((1, FC), lambda i: (0, 0)),
            pl.BlockSpec((TM4, CODE), lambda i: (i, 0)),
        ],
        out_shape=[
            jax.ShapeDtypeStruct((n, FC), jnp.float32),
            jax.ShapeDtypeStruct((1, FC), jnp.float32),
            jax.ShapeDtypeStruct((1, FC), jnp.float32),
            jax.ShapeDtypeStruct((n, CODE), jnp.float32),
        ],
    )(zqp, w)


_H5 = HALF // TM5


def _dec2_body(y0_ref, y1_ref, s0_ref, q0_ref, s1_ref, q1_ref, g_ref, b_ref,
               w_ref, b2_ref, o_ref):
    i = pl.program_id(0)
    y = jnp.where(i < _H5, y0_ref[...], y1_ref[...])
    mean = (s0_ref[...] + s1_ref[...]) * _INV_B
    var = (q0_ref[...] + q1_ref[...]) * _INV_B - mean * mean
    yn = g_ref[...] * (y - mean) * jax.lax.rsqrt(var + 1e-5) + b_ref[...]
    yn = jnp.where(yn >= 0, yn, 0.01 * yn)
    o_ref[...] = _mm(yn, w_ref[...]) + b2_ref[...]


def _decode2(y0, y1, s0, q0, s1, q1, g, b, w, b2):
    return pl.pallas_call(
        _dec2_body,
        grid=(BATCH // TM5,),
        in_specs=[
            pl.BlockSpec((TM5, FC), lambda i: (jnp.minimum(i, _H5 - 1), 0)),
            pl.BlockSpec((TM5, FC), lambda i: (jnp.maximum(i - _H5, 0), 0)),
            pl.BlockSpec((1, FC), lambda i: (0, 0)),
            pl.BlockSpec((1, FC), lambda i: (0, 0)),
            pl.BlockSpec((1, FC), lambda i: (0, 0)),
            pl.BlockSpec((1, FC), lambda i: (0, 0)),
            pl.BlockSpec((1, FC), lambda i: (0, 0)),
            pl.BlockSpec((1, FC), lambda i: (0, 0)),
            pl.BlockSpec((HIDDEN, FC), lambda i: (0, 0)),
            pl.BlockSpec((1, HIDDEN), lambda i: (0, 0)),
        ],
        out_specs=pl.BlockSpec((TM5, HIDDEN), lambda i: (i, 0)),
        out_shape=jax.ShapeDtypeStruct((BATCH, HIDDEN), jnp.float32),
    )(y0, y1, s0, q0, s1, q1, g, b, w, b2)


def kernel(h, W_e1, g_e1, be_e1, W_e2, b_e2, W_d1, g_d1, be_d1, W_d2, b_d2,
           codebook):
    x, xs, cn_col, cbp = _encode1(h, W_e1, codebook)
    xv = _var_pass(x, xs)
    cn_row = cn_col.reshape(1, -1)
    g1, b1 = g_e1.reshape(1, -1), be_e1.reshape(1, -1)
    b2r = b_e2.reshape(1, -1)
    ze0, idx0, idxrow0 = _encode2(x, xs, xv, g1, b1, W_e2, b2r, codebook,
                                  cn_row, 0)
    zqp0 = _gather_zq(cbp, idxrow0)
    ze1, idx1, idxrow1 = _encode2(x, xs, xv, g1, b1, W_e2, b2r, codebook,
                                  cn_row, 1)
    zqp1 = _gather_zq(cbp, idxrow1)
    y0, s0, q0, zq0 = _decode1(zqp0, W_d1)
    y1, s1, q1, zq1 = _decode1(zqp1, W_d1)
    h_recon = _decode2(y0, y1, s0, q0, s1, q1, g_d1.reshape(1, -1),
                       be_d1.reshape(1, -1), W_d2, b_d2.reshape(1, -1))
    ze = jnp.concatenate([ze0, ze1], axis=0)
    zq = jnp.concatenate([zq0, zq1], axis=0)
    idx = jnp.concatenate([idx0, idx1], axis=0).reshape(-1)
    return ze, zq, h_recon, idx
